# Initial kernel scaffold; baseline (speedup 1.0000x reference)
import jax
import jax.numpy as jnp
from jax.experimental import pallas as pl


def kernel(x, edge_index, edge_label_index, W1, a1s, a1d, b1, W2, a2s, a2d, b2, Wp1, bp1, g1, be1, Wp2, bp2, g2, be2, Wp3, bp3):
    raise NotImplementedError("write your pallas kernel here")


# speedup vs baseline: 1.0825x; 1.0825x over previous
import functools

import jax
import jax.numpy as jnp
from jax.experimental import pallas as pl
from jax.experimental.pallas import tpu as pltpu

N = 10000
E = 320000
EQ = 65536
F_IN = 128
HID = 256
OUT = 128
HEADS = 4

MLP_BLK = 1024


def _mlp_body(ef_ref, Wp1_ref, bp1_ref, s1_ref, be1_ref, Wp2_ref, bp2_ref,
              s2_ref, be2_ref, Wp3_ref, bp3_ref, out_ref):
    ef = ef_ref[...]
    h = jnp.dot(ef, Wp1_ref[...], preferred_element_type=jnp.float32)
    h = (h + bp1_ref[...]) * s1_ref[...] + be1_ref[...]
    h = jnp.maximum(h, 0.0)
    h2 = jnp.dot(h, Wp2_ref[...], preferred_element_type=jnp.float32)
    h2 = (h2 + bp2_ref[...]) * s2_ref[...] + be2_ref[...]
    h2 = jnp.maximum(h2, 0.0)
    o = jnp.dot(h2, Wp3_ref[...], preferred_element_type=jnp.float32)
    out_ref[...] = o + bp3_ref[...]


def _decode_mlp(ef, Wp1, bp1, g1, be1, Wp2, bp2, g2, be2, Wp3, bp3):
    inv = 1.0 / jnp.sqrt(jnp.float32(1.0 + 1e-5))
    s1 = (g1 * inv).reshape(1, HID)
    s2 = (g2 * inv).reshape(1, 32)
    grid = (EQ // MLP_BLK,)
    out = pl.pallas_call(
        _mlp_body,
        grid=grid,
        in_specs=[
            pl.BlockSpec((MLP_BLK, 2 * OUT), lambda i: (i, 0)),
            pl.BlockSpec((2 * OUT, HID), lambda i: (0, 0)),
            pl.BlockSpec((1, HID), lambda i: (0, 0)),
            pl.BlockSpec((1, HID), lambda i: (0, 0)),
            pl.BlockSpec((1, HID), lambda i: (0, 0)),
            pl.BlockSpec((HID, 32), lambda i: (0, 0)),
            pl.BlockSpec((1, 32), lambda i: (0, 0)),
            pl.BlockSpec((1, 32), lambda i: (0, 0)),
            pl.BlockSpec((1, 32), lambda i: (0, 0)),
            pl.BlockSpec((32, 1), lambda i: (0, 0)),
            pl.BlockSpec((1, 1), lambda i: (0, 0)),
        ],
        out_specs=pl.BlockSpec((MLP_BLK, 1), lambda i: (i, 0)),
        out_shape=jax.ShapeDtypeStruct((EQ, 1), jnp.float32),
    )(ef, Wp1, bp1.reshape(1, HID), s1, be1.reshape(1, HID), Wp2,
      bp2.reshape(1, 32), s2, be2.reshape(1, 32), Wp3, bp3.reshape(1, 1))
    return out.reshape(EQ)


def _gat(x, src, dst, W, a_s, a_d, b, heads, C, concat, n):
    h = (x @ W).reshape(n, heads, C)
    es = jnp.sum(h * a_s[None], axis=-1)
    ed = jnp.sum(h * a_d[None], axis=-1)
    e = jax.nn.leaky_relu(es[src] + ed[dst], 0.2)
    p = jnp.exp(e)
    ssum = jax.ops.segment_sum(p, dst, num_segments=n)
    alpha = p / (ssum[dst] + 1e-16)
    out = jax.ops.segment_sum(h[src] * alpha[:, :, None], dst, num_segments=n)
    if concat:
        out = out.reshape(n, heads * C)
    else:
        out = out.mean(axis=1)
    return out + b


def kernel(x, edge_index, edge_label_index, W1, a1s, a1d, b1, W2, a2s, a2d,
           b2, Wp1, bp1, g1, be1, Wp2, bp2, g2, be2, Wp3, bp3):
    n = x.shape[0]
    src, dst = edge_index[0], edge_index[1]
    z = _gat(x, src, dst, W1, a1s, a1d, b1, HEADS, HID, True, n)
    z = jax.nn.elu(z)
    z = _gat(z, src, dst, W2, a2s, a2d, b2, 1, OUT, False, n)
    row, col = edge_label_index[0], edge_label_index[1]
    ef = jnp.concatenate([z[row], z[col]], axis=-1)
    return _decode_mlp(ef, Wp1, bp1, g1, be1, Wp2, bp2, g2, be2, Wp3, bp3)


# speedup vs baseline: 16.2060x; 14.9712x over previous
import functools

import jax
import jax.numpy as jnp
from jax import lax
from jax.experimental import pallas as pl
from jax.experimental.pallas import tpu as pltpu
from jax.experimental.pallas import tpu_sc as plsc

N = 10000
NP = 10240
E = 320000
EQ = 65536
F_IN = 128
HID = 256
OUT = 128
HEADS = 4

NBLK = 400
MLP_BLK = 1024
K1 = 80
KD = 64
RPT = N // 16
CPT = NP // 16


def _tc1_body(x_ref, w1_ref, acat_ref, t1_ref, esed_ref):
    h1 = jnp.dot(x_ref[...], w1_ref[...], preferred_element_type=jnp.float32)
    for p in range(8):
        t1_ref[p] = h1[:, p * 128:(p + 1) * 128]
    esed_ref[...] = jnp.dot(h1, acat_ref[...],
                            preferred_element_type=jnp.float32)


def _tc2_body(outp_ref, ssum_ref, b1_ref, w2_ref, a2cat_ref, t2_ref,
              esed2_ref):
    acc = jnp.zeros((NBLK, OUT), jnp.float32)
    for p in range(8):
        h = p // 2
        denom = ssum_ref[:, h:h + 1] + 1e-16
        z = outp_ref[p] / denom + b1_ref[0:1, p * 128:(p + 1) * 128]
        z = jnp.where(z > 0, z, jnp.exp(jnp.minimum(z, 0.0)) - 1.0)
        acc = acc + jnp.dot(z, w2_ref[p * 128:(p + 1) * 128, :],
                            preferred_element_type=jnp.float32)
    t2_ref[...] = acc
    esed2_ref[...] = jnp.dot(acc, a2cat_ref[...],
                             preferred_element_type=jnp.float32)


def _tc3_body(out2p_ref, ssum2_ref, b2_ref, wp1_ref, u_ref, v_ref):
    denom = ssum2_ref[:, 0:1] + ssum2_ref[:, 1:2] + 1e-16
    z = (out2p_ref[0] + out2p_ref[1]) / denom + b2_ref[...]
    u_ref[...] = jnp.dot(z, wp1_ref[:OUT, :],
                         preferred_element_type=jnp.float32)
    v_ref[...] = jnp.dot(z, wp1_ref[OUT:, :],
                         preferred_element_type=jnp.float32)


def _tc4_body(ef_ref, bp1_ref, s1_ref, be1_ref, wp2_ref, bp2_ref, s2_ref,
              be2_ref, wp3_ref, bp3_ref, out_ref):
    h = (ef_ref[...] + bp1_ref[...]) * s1_ref[...] + be1_ref[...]
    h = jnp.maximum(h, 0.0)
    h2 = jnp.dot(h, wp2_ref[...], preferred_element_type=jnp.float32)
    h2 = (h2 + bp2_ref[...]) * s2_ref[...] + be2_ref[...]
    h2 = jnp.maximum(h2, 0.0)
    o = jnp.dot(h2, wp3_ref[...], preferred_element_type=jnp.float32)
    out_ref[...] = o + bp3_ref[...]


_SC_MESH = plsc.VectorSubcoreMesh(core_axis_name="c", subcore_axis_name="s")


def _zero_vec_ref(ref, nvec):
    z = jnp.zeros((16,), jnp.float32)

    def zb(i, _):
        ref[pl.ds(i * 16, 16)] = z
        return 0

    lax.fori_loop(0, nvec, zb, 0)


def _scale_rows(rows, pv, nrows, width_vecs):
    def sb(i, _):
        scv = plsc.load_gather(pv, [jnp.full((16,), 0, jnp.int32) + i])
        for v in range(width_vecs):
            sl = pl.ds(v * 16, 16)
            rows[i, sl] = rows[i, sl] * scv
        return 0

    lax.fori_loop(0, nrows, sb, 0)


def _tree_reduce_ssum(slab_sh, s, ssum_v, tmp_v, out_hbm, out_off):
    for r in (8, 4, 2, 1):
        @pl.when((s >= r) & (s < 2 * r))
        def _():
            pltpu.sync_copy(ssum_v, slab_sh.at[s - r])

        plsc.subcore_barrier()

        @pl.when(s < r)
        def _():
            pltpu.sync_copy(slab_sh.at[s], tmp_v)

            def ab(i, _):
                sl = pl.ds(i * 16, 16)
                ssum_v[sl] = ssum_v[sl] + tmp_v[sl]
                return 0

            lax.fori_loop(0, N // 16, ab, 0)

        plsc.subcore_barrier()

    @pl.when(s == 0)
    def _():
        pltpu.sync_copy(ssum_v, out_hbm.at[pl.ds(out_off, N)])


def _sc_l1_body(t1_hbm, esed_hbm, src_hbm, dst_hbm, zer_hbm,
                out_hbm, ssum_hbm,
                es_v, ed_v, ssum_v, src_v, dst_v, pv, gidx, didx, rows,
                sem, acc_sh, slab_sh):
    c = lax.axis_index("c")
    s = lax.axis_index("s")

    def one_pass(pp, _):
        p = c * 4 + pp
        h = c * 2 + pp // 2
        even = (pp % 2) == 0
        pltpu.sync_copy(esed_hbm.at[pl.ds(h * N, N)], es_v)
        pltpu.sync_copy(esed_hbm.at[pl.ds((4 + h) * N, N)], ed_v)
        pltpu.sync_copy(zer_hbm, acc_sh.at[pl.ds(s * RPT, RPT)])

        @pl.when(even)
        def _():
            _zero_vec_ref(ssum_v, N // 16)

        plsc.subcore_barrier()

        def one_block(blk, _):
            e0 = s * 20000 + blk * 2000
            pltpu.sync_copy(src_hbm.at[pl.ds(e0, 2000)], src_v)
            pltpu.sync_copy(dst_hbm.at[pl.ds(e0, 2000)], dst_v)

            def one_batch(kb, _):
                o = kb * K1
                for j in range(K1 // 16):
                    sl = pl.ds(o + j * 16, 16)
                    bsl = pl.ds(j * 16, 16)
                    s16 = src_v[sl]
                    gidx[bsl] = s16 + p * N
                    didx[bsl] = dst_v[sl]
                cp = pltpu.async_copy(t1_hbm.at[gidx], rows, sem)
                for j in range(K1 // 16):
                    sl = pl.ds(o + j * 16, 16)
                    bsl = pl.ds(j * 16, 16)
                    ev = plsc.load_gather(es_v, [src_v[sl]])
                    dv = plsc.load_gather(ed_v, [dst_v[sl]])
                    t = ev + dv
                    e = jnp.where(t > 0, t, t * 0.2)
                    pvec = jnp.exp(e)
                    pv[bsl] = pvec

                    @pl.when(even)
                    def _():
                        plsc.addupdate_scatter(ssum_v, [dst_v[sl]], pvec)

                cp.wait()
                _scale_rows(rows, pv, K1, 8)
                pltpu.sync_copy(rows, acc_sh.at[didx], add=True)
                return 0

            lax.fori_loop(0, 25, one_batch, 0)
            return 0

        lax.fori_loop(0, 10, one_block, 0)
        plsc.subcore_barrier()
        pltpu.sync_copy(acc_sh.at[pl.ds(s * RPT, RPT)],
                        out_hbm.at[pl.ds(p * N + s * RPT, RPT)])

        @pl.when(even)
        def _():
            _tree_reduce_ssum(slab_sh, s, ssum_v, es_v, ssum_hbm, h * N)

        plsc.subcore_barrier()
        return 0

    lax.fori_loop(0, 4, one_pass, 0)


def _sc_l2_body(t2_hbm, esed2_hbm, src_hbm, dst_hbm, zer_hbm,
                out_hbm, ssum_hbm,
                es_v, ed_v, ssum_v, src_v, dst_v, pv, gidx, didx, rows,
                sem, acc_sh, slab_sh):
    c = lax.axis_index("c")
    s = lax.axis_index("s")
    pltpu.sync_copy(esed2_hbm.at[pl.ds(0, N)], es_v)
    pltpu.sync_copy(esed2_hbm.at[pl.ds(N, N)], ed_v)
    pltpu.sync_copy(zer_hbm, acc_sh.at[pl.ds(s * RPT, RPT)])
    _zero_vec_ref(ssum_v, N // 16)
    plsc.subcore_barrier()
    base = (c * 16 + s) * 10000

    def one_block(blk, _):
        e0 = base + blk * 2000
        pltpu.sync_copy(src_hbm.at[pl.ds(e0, 2000)], src_v)
        pltpu.sync_copy(dst_hbm.at[pl.ds(e0, 2000)], dst_v)

        def one_batch(kb, _):
            o = kb * K1
            for j in range(K1 // 16):
                sl = pl.ds(o + j * 16, 16)
                bsl = pl.ds(j * 16, 16)
                gidx[bsl] = src_v[sl]
                didx[bsl] = dst_v[sl]
            cp = pltpu.async_copy(t2_hbm.at[gidx], rows, sem)
            for j in range(K1 // 16):
                sl = pl.ds(o + j * 16, 16)
                bsl = pl.ds(j * 16, 16)
                ev = plsc.load_gather(es_v, [src_v[sl]])
                dv = plsc.load_gather(ed_v, [dst_v[sl]])
                t = ev + dv
                e = jnp.where(t > 0, t, t * 0.2)
                pvec = jnp.exp(e)
                pv[bsl] = pvec
                plsc.addupdate_scatter(ssum_v, [dst_v[sl]], pvec)
            cp.wait()
            _scale_rows(rows, pv, K1, 8)
            pltpu.sync_copy(rows, acc_sh.at[didx], add=True)
            return 0

        lax.fori_loop(0, 25, one_batch, 0)
        return 0

    lax.fori_loop(0, 5, one_block, 0)
    plsc.subcore_barrier()
    pltpu.sync_copy(acc_sh.at[pl.ds(s * RPT, RPT)],
                    out_hbm.at[pl.ds(c * N + s * RPT, RPT)])
    _tree_reduce_ssum(slab_sh, s, ssum_v, es_v, ssum_hbm, c * N)


def _sc_dec_body(u_hbm, v_hbm, row_hbm, col_hbm, ef_hbm,
                 row_v, col_v, ru, rv, semu, semv):
    c = lax.axis_index("c")
    s = lax.axis_index("s")
    tid = c * 16 + s
    q0 = tid * (EQ // 32)
    pltpu.sync_copy(row_hbm.at[pl.ds(q0, EQ // 32)], row_v)
    pltpu.sync_copy(col_hbm.at[pl.ds(q0, EQ // 32)], col_v)

    def one_batch(kb, _):
        o = kb * KD
        cu = pltpu.async_copy(u_hbm.at[row_v.at[pl.ds(o, KD)]], ru, semu)
        cv = pltpu.async_copy(v_hbm.at[col_v.at[pl.ds(o, KD)]], rv, semv)
        cu.wait()
        cv.wait()

        def add_row(i, _):
            for v in range(16):
                sl = pl.ds(v * 16, 16)
                ru[i, sl] = ru[i, sl] + rv[i, sl]
            return 0

        lax.fori_loop(0, KD, add_row, 0)
        pltpu.sync_copy(ru, ef_hbm.at[pl.ds(q0 + o, KD)])
        return 0

    lax.fori_loop(0, (EQ // 32) // KD, one_batch, 0)


_sc_l1 = functools.partial(
    pl.kernel, _sc_l1_body,
    out_type=(jax.ShapeDtypeStruct((8 * N, 128), jnp.float32),
              jax.ShapeDtypeStruct((4 * N,), jnp.float32)),
    mesh=_SC_MESH,
    compiler_params=pltpu.CompilerParams(use_tc_tiling_on_sc=False, needs_layout_passes=False),
    scratch_types=[
        pltpu.VMEM((N,), jnp.float32),
        pltpu.VMEM((N,), jnp.float32),
        pltpu.VMEM((N,), jnp.float32),
        pltpu.VMEM((2000,), jnp.int32),
        pltpu.VMEM((2000,), jnp.int32),
        pltpu.VMEM((K1,), jnp.float32),
        pltpu.VMEM((K1,), jnp.int32),
        pltpu.VMEM((K1,), jnp.int32),
        pltpu.VMEM((K1, 128), jnp.float32),
        pltpu.SemaphoreType.DMA,
        pltpu.VMEM_SHARED((N, 128), jnp.float32),
        pltpu.VMEM_SHARED((8, N), jnp.float32),
    ])

_sc_l2 = functools.partial(
    pl.kernel, _sc_l2_body,
    out_type=(jax.ShapeDtypeStruct((2 * N, 128), jnp.float32),
              jax.ShapeDtypeStruct((2 * N,), jnp.float32)),
    mesh=_SC_MESH,
    compiler_params=pltpu.CompilerParams(use_tc_tiling_on_sc=False, needs_layout_passes=False),
    scratch_types=[
        pltpu.VMEM((N,), jnp.float32),
        pltpu.VMEM((N,), jnp.float32),
        pltpu.VMEM((N,), jnp.float32),
        pltpu.VMEM((2000,), jnp.int32),
        pltpu.VMEM((2000,), jnp.int32),
        pltpu.VMEM((K1,), jnp.float32),
        pltpu.VMEM((K1,), jnp.int32),
        pltpu.VMEM((K1,), jnp.int32),
        pltpu.VMEM((K1, 128), jnp.float32),
        pltpu.SemaphoreType.DMA,
        pltpu.VMEM_SHARED((N, 128), jnp.float32),
        pltpu.VMEM_SHARED((8, N), jnp.float32),
    ])

_sc_dec = functools.partial(
    pl.kernel, _sc_dec_body,
    out_type=jax.ShapeDtypeStruct((EQ, 2 * OUT), jnp.float32),
    mesh=_SC_MESH,
    compiler_params=pltpu.CompilerParams(use_tc_tiling_on_sc=False, needs_layout_passes=False),
    scratch_types=[
        pltpu.VMEM((EQ // 32,), jnp.int32),
        pltpu.VMEM((EQ // 32,), jnp.int32),
        pltpu.VMEM((KD, 2 * OUT), jnp.float32),
        pltpu.VMEM((KD, 2 * OUT), jnp.float32),
        pltpu.SemaphoreType.DMA,
        pltpu.SemaphoreType.DMA,
    ])


def kernel(x, edge_index, edge_label_index, W1, a1s, a1d, b1, W2, a2s, a2d,
           b2, Wp1, bp1, g1, be1, Wp2, bp2, g2, be2, Wp3, bp3):
    src = edge_index[0].astype(jnp.int32)
    dst = edge_index[1].astype(jnp.int32)
    row = edge_label_index[0].astype(jnp.int32)
    col = edge_label_index[1].astype(jnp.int32)

    eye4 = jnp.eye(HEADS, dtype=jnp.float32)
    asb = (eye4[:, None, :] * a1s[:, :, None]).reshape(HEADS * HID, HEADS)
    adb = (eye4[:, None, :] * a1d[:, :, None]).reshape(HEADS * HID, HEADS)
    acat = jnp.concatenate([asb, adb], axis=1)
    a2cat = jnp.concatenate([a2s, a2d], axis=0).T
    zer = jnp.zeros((RPT, 128), jnp.float32)

    t1, esed1 = pl.pallas_call(
        _tc1_body,
        grid=(N // NBLK,),
        in_specs=[
            pl.BlockSpec((NBLK, F_IN), lambda i: (i, 0)),
            pl.BlockSpec((F_IN, HEADS * HID), lambda i: (0, 0)),
            pl.BlockSpec((HEADS * HID, 8), lambda i: (0, 0)),
        ],
        out_specs=[
            pl.BlockSpec((8, NBLK, 128), lambda i: (0, i, 0)),
            pl.BlockSpec((NBLK, 8), lambda i: (i, 0)),
        ],
        out_shape=[
            jax.ShapeDtypeStruct((8, N, 128), jnp.float32),
            jax.ShapeDtypeStruct((N, 8), jnp.float32),
        ],
    )(x, W1, acat)

    out1p, ssum1 = _sc_l1()(t1.reshape(8 * N, 128), esed1.T.reshape(8 * N),
                            src, dst, zer)
    ssum1t = ssum1.reshape(4, N).T

    t2, esed2 = pl.pallas_call(
        _tc2_body,
        grid=(N // NBLK,),
        in_specs=[
            pl.BlockSpec((8, NBLK, 128), lambda i: (0, i, 0)),
            pl.BlockSpec((NBLK, HEADS), lambda i: (i, 0)),
            pl.BlockSpec((1, HEADS * HID), lambda i: (0, 0)),
            pl.BlockSpec((HEADS * HID, OUT), lambda i: (0, 0)),
            pl.BlockSpec((OUT, 2), lambda i: (0, 0)),
        ],
        out_specs=[
            pl.BlockSpec((NBLK, OUT), lambda i: (i, 0)),
            pl.BlockSpec((NBLK, 2), lambda i: (i, 0)),
        ],
        out_shape=[
            jax.ShapeDtypeStruct((N, OUT), jnp.float32),
            jax.ShapeDtypeStruct((N, 2), jnp.float32),
        ],
    )(out1p.reshape(8, N, 128), ssum1t, b1.reshape(1, HEADS * HID), W2,
      a2cat)

    out2p, ssum2 = _sc_l2()(t2, esed2.T.reshape(2 * N), src, dst, zer)
    ssum2t = ssum2.reshape(2, N).T

    u, v = pl.pallas_call(
        _tc3_body,
        grid=(N // NBLK,),
        in_specs=[
            pl.BlockSpec((2, NBLK, OUT), lambda i: (0, i, 0)),
            pl.BlockSpec((NBLK, 2), lambda i: (i, 0)),
            pl.BlockSpec((1, OUT), lambda i: (0, 0)),
            pl.BlockSpec((2 * OUT, HID), lambda i: (0, 0)),
        ],
        out_specs=[
            pl.BlockSpec((NBLK, HID), lambda i: (i, 0)),
            pl.BlockSpec((NBLK, HID), lambda i: (i, 0)),
        ],
        out_shape=[
            jax.ShapeDtypeStruct((N, HID), jnp.float32),
            jax.ShapeDtypeStruct((N, HID), jnp.float32),
        ],
    )(out2p.reshape(2, N, 128), ssum2t, b2.reshape(1, OUT), Wp1)

    ef = _sc_dec()(u, v, row, col)

    inv = 1.0 / jnp.sqrt(jnp.float32(1.0 + 1e-5))
    s1 = (g1 * inv).reshape(1, HID)
    s2 = (g2 * inv).reshape(1, 32)
    out = pl.pallas_call(
        _tc4_body,
        grid=(EQ // MLP_BLK,),
        in_specs=[
            pl.BlockSpec((MLP_BLK, 2 * OUT), lambda i: (i, 0)),
            pl.BlockSpec((1, HID), lambda i: (0, 0)),
            pl.BlockSpec((1, HID), lambda i: (0, 0)),
            pl.BlockSpec((1, HID), lambda i: (0, 0)),
            pl.BlockSpec((HID, 32), lambda i: (0, 0)),
            pl.BlockSpec((1, 32), lambda i: (0, 0)),
            pl.BlockSpec((1, 32), lambda i: (0, 0)),
            pl.BlockSpec((1, 32), lambda i: (0, 0)),
            pl.BlockSpec((32, 1), lambda i: (0, 0)),
            pl.BlockSpec((1, 1), lambda i: (0, 0)),
        ],
        out_specs=pl.BlockSpec((MLP_BLK, 1), lambda i: (i, 0)),
        out_shape=jax.ShapeDtypeStruct((EQ, 1), jnp.float32),
    )(ef, bp1.reshape(1, HID), s1, be1.reshape(1, HID), Wp2,
      bp2.reshape(1, 32), s2, be2.reshape(1, 32), Wp3, bp3.reshape(1, 1))
    return out.reshape(EQ)


# speedup vs baseline: 25.4916x; 1.5730x over previous
import functools

import jax
import jax.numpy as jnp
from jax import lax
from jax.experimental import pallas as pl
from jax.experimental.pallas import tpu as pltpu
from jax.experimental.pallas import tpu_sc as plsc

N = 10000
NP = 10240
E = 320000
EQ = 65536
F_IN = 128
HID = 256
OUT = 128
HEADS = 4

NBLK = 400
MLP_BLK = 1024
K1 = 80
KD = 64
RPT = N // 16
CPT = NP // 16


def _tc1_body(x_ref, w1_ref, acat_ref, t1_ref, esed_ref):
    h1 = jnp.dot(x_ref[...], w1_ref[...], preferred_element_type=jnp.float32)
    for p in range(8):
        t1_ref[p] = h1[:, p * 128:(p + 1) * 128]
    esed_ref[...] = jnp.dot(h1, acat_ref[...],
                            preferred_element_type=jnp.float32)


def _tc2_body(outp_ref, ssum_ref, b1_ref, w2_ref, a2cat_ref, t2_ref,
              esed2_ref):
    acc = jnp.zeros((NBLK, OUT), jnp.float32)
    for p in range(8):
        h = p // 2
        denom = ssum_ref[:, h:h + 1] + 1e-16
        z = outp_ref[p] / denom + b1_ref[0:1, p * 128:(p + 1) * 128]
        z = jnp.where(z > 0, z, jnp.exp(jnp.minimum(z, 0.0)) - 1.0)
        acc = acc + jnp.dot(z, w2_ref[p * 128:(p + 1) * 128, :],
                            preferred_element_type=jnp.float32)
    t2_ref[...] = acc
    esed2_ref[...] = jnp.dot(acc, a2cat_ref[...],
                             preferred_element_type=jnp.float32)


def _tc3_body(out2p_ref, ssum2_ref, b2_ref, wp1_ref, u_ref, v_ref):
    denom = ssum2_ref[:, 0:1] + ssum2_ref[:, 1:2] + 1e-16
    z = (out2p_ref[0] + out2p_ref[1]) / denom + b2_ref[...]
    u_ref[...] = jnp.dot(z, wp1_ref[:OUT, :],
                         preferred_element_type=jnp.float32)
    v_ref[...] = jnp.dot(z, wp1_ref[OUT:, :],
                         preferred_element_type=jnp.float32)


def _tc4_body(ef_ref, bp1_ref, s1_ref, be1_ref, wp2_ref, bp2_ref, s2_ref,
              be2_ref, wp3_ref, bp3_ref, out_ref):
    h = (ef_ref[...] + bp1_ref[...]) * s1_ref[...] + be1_ref[...]
    h = jnp.maximum(h, 0.0)
    h2 = jnp.dot(h, wp2_ref[...], preferred_element_type=jnp.float32)
    h2 = (h2 + bp2_ref[...]) * s2_ref[...] + be2_ref[...]
    h2 = jnp.maximum(h2, 0.0)
    o = jnp.dot(h2, wp3_ref[...], preferred_element_type=jnp.float32)
    out_ref[...] = o + bp3_ref[...]


_SC_MESH = plsc.VectorSubcoreMesh(core_axis_name="c", subcore_axis_name="s")


def _zero_vec_ref(ref, nvec):
    z = jnp.zeros((16,), jnp.float32)

    def zb(i, _):
        ref[pl.ds(i * 16, 16)] = z
        return 0

    lax.fori_loop(0, nvec, zb, 0)


def _scale_rows(rows, pv, nrows, width_vecs):
    def sb(i, _):
        scv = plsc.load_gather(pv, [jnp.full((16,), 0, jnp.int32) + i])
        for v in range(width_vecs):
            sl = pl.ds(v * 16, 16)
            rows[i, sl] = rows[i, sl] * scv
        return 0

    lax.fori_loop(0, nrows, sb, 0)


def _edge_pass(nblk, do_ssum, base_e, tab_base, t_hbm, src_hbm, dst_hbm,
               es_v, ed_v, src_v, dst_v, pv, gidx, didx, rows,
               semb, semg, semsc, semp, acc_sh, ssum_sh):
    G = nblk * 25

    def stage_block(b, buf):
        e0 = base_e + b * 2000
        pltpu.async_copy(src_hbm.at[pl.ds(e0, 2000)], src_v.at[buf],
                         semb.at[buf])
        pltpu.async_copy(dst_hbm.at[pl.ds(e0, 2000)], dst_v.at[buf],
                         semb.at[buf])

    def wait_block(b, buf):
        e0 = base_e + b * 2000
        pltpu.make_async_copy(src_hbm.at[pl.ds(e0, 2000)], src_v.at[buf],
                              semb.at[buf]).wait()
        pltpu.make_async_copy(dst_hbm.at[pl.ds(e0, 2000)], dst_v.at[buf],
                              semb.at[buf]).wait()

    def scalars(gg, buf):
        bb = (gg // 25) % 2
        o = (gg % 25) * K1
        for j in range(K1 // 16):
            sl = pl.ds(o + j * 16, 16)
            bsl = pl.ds(j * 16, 16)
            s16 = src_v[bb, sl]
            d16 = dst_v[bb, sl]
            gidx[buf, bsl] = s16 + tab_base
            didx[buf, bsl] = d16
            ev = plsc.load_gather(es_v, [s16])
            dv = plsc.load_gather(ed_v, [d16])
            t = ev + dv
            e = jnp.where(t > 0, t, t * 0.2)
            pv[buf, bsl] = jnp.exp(e)

    def issue_gather(buf):
        pltpu.async_copy(t_hbm.at[gidx.at[buf]], rows.at[buf], semg.at[buf])

    def wait_gather(buf):
        pltpu.make_async_copy(t_hbm.at[gidx.at[buf]], rows.at[buf],
                              semg.at[buf]).wait()

    def issue_scatter(buf):
        pltpu.async_copy(rows.at[buf], acc_sh.at[didx.at[buf]],
                         semsc.at[buf], add=True)

        @pl.when(do_ssum)
        def _():
            pltpu.async_copy(pv.at[buf], ssum_sh.at[didx.at[buf]],
                             semp.at[buf], add=True)

    def wait_scatter(buf):
        pltpu.make_async_copy(rows.at[buf], acc_sh.at[didx.at[buf]],
                              semsc.at[buf]).wait()

        @pl.when(do_ssum)
        def _():
            pltpu.make_async_copy(pv.at[buf], ssum_sh.at[didx.at[buf]],
                                  semp.at[buf]).wait()

    def scale(buf):
        def sb(i, _):
            b16 = jnp.full((16,), 0, jnp.int32) + buf
            i16 = jnp.full((16,), 0, jnp.int32) + i
            scv = plsc.load_gather(pv, [b16, i16])
            for v in range(8):
                sl = pl.ds(v * 16, 16)
                rows[buf, i, sl] = rows[buf, i, sl] * scv
            return 0

        lax.fori_loop(0, K1, sb, 0)

    stage_block(0, 0)
    wait_block(0, 0)
    if nblk > 1:
        stage_block(1, 1)
    scalars(0, 0)
    issue_gather(0)

    def it(g, _):
        buf = g % 2
        buf2 = (g + 1) % 2
        wait_gather(buf)

        @pl.when(g + 1 < G)
        def _():
            @pl.when((g + 1) % 25 == 0)
            def _():
                b = (g + 1) // 25
                wait_block(b, b % 2)

                @pl.when(b + 1 < nblk)
                def _():
                    stage_block(b + 1, (b + 1) % 2)

            @pl.when(g >= 1)
            def _():
                wait_scatter(buf2)

            scalars(g + 1, buf2)
            issue_gather(buf2)

        scale(buf)
        issue_scatter(buf)
        return 0

    lax.fori_loop(0, G, it, 0)
    wait_scatter(0)
    wait_scatter(1)


def _sc_l1_body(t1_hbm, esed_hbm, src_hbm, dst_hbm, zer_hbm, zer1_hbm,
                out_hbm, ssum_hbm,
                es_v, ed_v, src_v, dst_v, pv, gidx, didx, rows,
                semb, semg, semsc, semp, acc_sh, ssum_sh):
    c = lax.axis_index("c")
    s = lax.axis_index("s")

    def one_pass(pp, _):
        p = c * 4 + pp
        h = c * 2 + pp // 2
        even = (pp % 2) == 0
        pltpu.sync_copy(esed_hbm.at[pl.ds(h * N, N)], es_v)
        pltpu.sync_copy(esed_hbm.at[pl.ds((4 + h) * N, N)], ed_v)
        pltpu.sync_copy(zer_hbm, acc_sh.at[pl.ds(s * RPT, RPT)])

        @pl.when(even)
        def _():
            pltpu.sync_copy(zer1_hbm, ssum_sh.at[pl.ds(s * CPT, CPT)])

        plsc.subcore_barrier()
        _edge_pass(10, even, s * 20000, p * N, t1_hbm, src_hbm, dst_hbm,
                   es_v, ed_v, src_v, dst_v, pv, gidx, didx, rows,
                   semb, semg, semsc, semp, acc_sh, ssum_sh)
        plsc.subcore_barrier()
        pltpu.sync_copy(acc_sh.at[pl.ds(s * RPT, RPT)],
                        out_hbm.at[pl.ds(p * N + s * RPT, RPT)])

        @pl.when(even)
        def _():
            pltpu.sync_copy(ssum_sh.at[pl.ds(s * CPT, CPT)],
                            ssum_hbm.at[pl.ds(h * NP + s * CPT, CPT)])

        plsc.subcore_barrier()
        return 0

    lax.fori_loop(0, 4, one_pass, 0)


def _sc_l2_body(t2_hbm, esed2_hbm, src_hbm, dst_hbm, zer_hbm, zer1_hbm,
                out_hbm, ssum_hbm,
                es_v, ed_v, src_v, dst_v, pv, gidx, didx, rows,
                semb, semg, semsc, semp, acc_sh, ssum_sh):
    c = lax.axis_index("c")
    s = lax.axis_index("s")
    pltpu.sync_copy(esed2_hbm.at[pl.ds(0, N)], es_v)
    pltpu.sync_copy(esed2_hbm.at[pl.ds(N, N)], ed_v)
    pltpu.sync_copy(zer_hbm, acc_sh.at[pl.ds(s * RPT, RPT)])
    pltpu.sync_copy(zer1_hbm, ssum_sh.at[pl.ds(s * CPT, CPT)])
    plsc.subcore_barrier()
    _edge_pass(5, s >= 0, (c * 16 + s) * 10000, 0, t2_hbm, src_hbm, dst_hbm,
               es_v, ed_v, src_v, dst_v, pv, gidx, didx, rows,
               semb, semg, semsc, semp, acc_sh, ssum_sh)
    plsc.subcore_barrier()
    pltpu.sync_copy(acc_sh.at[pl.ds(s * RPT, RPT)],
                    out_hbm.at[pl.ds(c * N + s * RPT, RPT)])
    pltpu.sync_copy(ssum_sh.at[pl.ds(s * CPT, CPT)],
                    ssum_hbm.at[pl.ds(c * NP + s * CPT, CPT)])


def _sc_dec_body(u_hbm, v_hbm, row_hbm, col_hbm, ef_hbm,
                 row_v, col_v, ru, rv, semu, semv):
    c = lax.axis_index("c")
    s = lax.axis_index("s")
    tid = c * 16 + s
    q0 = tid * (EQ // 32)
    pltpu.sync_copy(row_hbm.at[pl.ds(q0, EQ // 32)], row_v)
    pltpu.sync_copy(col_hbm.at[pl.ds(q0, EQ // 32)], col_v)

    def one_batch(kb, _):
        o = kb * KD
        cu = pltpu.async_copy(u_hbm.at[row_v.at[pl.ds(o, KD)]], ru, semu)
        cv = pltpu.async_copy(v_hbm.at[col_v.at[pl.ds(o, KD)]], rv, semv)
        cu.wait()
        cv.wait()

        def add_row(i, _):
            for v in range(16):
                sl = pl.ds(v * 16, 16)
                ru[i, sl] = ru[i, sl] + rv[i, sl]
            return 0

        lax.fori_loop(0, KD, add_row, 0)
        pltpu.sync_copy(ru, ef_hbm.at[pl.ds(q0 + o, KD)])
        return 0

    lax.fori_loop(0, (EQ // 32) // KD, one_batch, 0)


_sc_l1 = functools.partial(
    pl.kernel, _sc_l1_body,
    out_type=(jax.ShapeDtypeStruct((8 * N, 128), jnp.float32),
              jax.ShapeDtypeStruct((4 * NP,), jnp.float32)),
    mesh=_SC_MESH,
    compiler_params=pltpu.CompilerParams(use_tc_tiling_on_sc=False, needs_layout_passes=False),
    scratch_types=[
        pltpu.VMEM((N,), jnp.float32),
        pltpu.VMEM((N,), jnp.float32),
        pltpu.VMEM((2, 2000), jnp.int32),
        pltpu.VMEM((2, 2000), jnp.int32),
        pltpu.VMEM((2, K1), jnp.float32),
        pltpu.VMEM((2, K1), jnp.int32),
        pltpu.VMEM((2, K1), jnp.int32),
        pltpu.VMEM((2, K1, 128), jnp.float32),
        pltpu.SemaphoreType.DMA((2,)),
        pltpu.SemaphoreType.DMA((2,)),
        pltpu.SemaphoreType.DMA((2,)),
        pltpu.SemaphoreType.DMA((2,)),
        pltpu.VMEM_SHARED((N, 128), jnp.float32),
        pltpu.VMEM_SHARED((NP,), jnp.float32),
    ])

_sc_l2 = functools.partial(
    pl.kernel, _sc_l2_body,
    out_type=(jax.ShapeDtypeStruct((2 * N, 128), jnp.float32),
              jax.ShapeDtypeStruct((2 * NP,), jnp.float32)),
    mesh=_SC_MESH,
    compiler_params=pltpu.CompilerParams(use_tc_tiling_on_sc=False, needs_layout_passes=False),
    scratch_types=[
        pltpu.VMEM((N,), jnp.float32),
        pltpu.VMEM((N,), jnp.float32),
        pltpu.VMEM((2, 2000), jnp.int32),
        pltpu.VMEM((2, 2000), jnp.int32),
        pltpu.VMEM((2, K1), jnp.float32),
        pltpu.VMEM((2, K1), jnp.int32),
        pltpu.VMEM((2, K1), jnp.int32),
        pltpu.VMEM((2, K1, 128), jnp.float32),
        pltpu.SemaphoreType.DMA((2,)),
        pltpu.SemaphoreType.DMA((2,)),
        pltpu.SemaphoreType.DMA((2,)),
        pltpu.SemaphoreType.DMA((2,)),
        pltpu.VMEM_SHARED((N, 128), jnp.float32),
        pltpu.VMEM_SHARED((NP,), jnp.float32),
    ])

_sc_dec = functools.partial(
    pl.kernel, _sc_dec_body,
    out_type=jax.ShapeDtypeStruct((EQ, 2 * OUT), jnp.float32),
    mesh=_SC_MESH,
    compiler_params=pltpu.CompilerParams(use_tc_tiling_on_sc=False, needs_layout_passes=False),
    scratch_types=[
        pltpu.VMEM((EQ // 32,), jnp.int32),
        pltpu.VMEM((EQ // 32,), jnp.int32),
        pltpu.VMEM((KD, 2 * OUT), jnp.float32),
        pltpu.VMEM((KD, 2 * OUT), jnp.float32),
        pltpu.SemaphoreType.DMA,
        pltpu.SemaphoreType.DMA,
    ])


def kernel(x, edge_index, edge_label_index, W1, a1s, a1d, b1, W2, a2s, a2d,
           b2, Wp1, bp1, g1, be1, Wp2, bp2, g2, be2, Wp3, bp3):
    src = edge_index[0].astype(jnp.int32)
    dst = edge_index[1].astype(jnp.int32)
    row = edge_label_index[0].astype(jnp.int32)
    col = edge_label_index[1].astype(jnp.int32)

    eye4 = jnp.eye(HEADS, dtype=jnp.float32)
    asb = (eye4[:, None, :] * a1s[:, :, None]).reshape(HEADS * HID, HEADS)
    adb = (eye4[:, None, :] * a1d[:, :, None]).reshape(HEADS * HID, HEADS)
    acat = jnp.concatenate([asb, adb], axis=1)
    a2cat = jnp.concatenate([a2s, a2d], axis=0).T
    zer = jnp.zeros((RPT, 128), jnp.float32)
    zer1 = jnp.zeros((CPT,), jnp.float32)

    t1, esed1 = pl.pallas_call(
        _tc1_body,
        grid=(N // NBLK,),
        in_specs=[
            pl.BlockSpec((NBLK, F_IN), lambda i: (i, 0)),
            pl.BlockSpec((F_IN, HEADS * HID), lambda i: (0, 0)),
            pl.BlockSpec((HEADS * HID, 8), lambda i: (0, 0)),
        ],
        out_specs=[
            pl.BlockSpec((8, NBLK, 128), lambda i: (0, i, 0)),
            pl.BlockSpec((NBLK, 8), lambda i: (i, 0)),
        ],
        out_shape=[
            jax.ShapeDtypeStruct((8, N, 128), jnp.float32),
            jax.ShapeDtypeStruct((N, 8), jnp.float32),
        ],
    )(x, W1, acat)

    out1p, ssum1 = _sc_l1()(t1.reshape(8 * N, 128), esed1.T.reshape(8 * N),
                            src, dst, zer, zer1)
    ssum1t = ssum1.reshape(4, NP)[:, :N].T

    t2, esed2 = pl.pallas_call(
        _tc2_body,
        grid=(N // NBLK,),
        in_specs=[
            pl.BlockSpec((8, NBLK, 128), lambda i: (0, i, 0)),
            pl.BlockSpec((NBLK, HEADS), lambda i: (i, 0)),
            pl.BlockSpec((1, HEADS * HID), lambda i: (0, 0)),
            pl.BlockSpec((HEADS * HID, OUT), lambda i: (0, 0)),
            pl.BlockSpec((OUT, 2), lambda i: (0, 0)),
        ],
        out_specs=[
            pl.BlockSpec((NBLK, OUT), lambda i: (i, 0)),
            pl.BlockSpec((NBLK, 2), lambda i: (i, 0)),
        ],
        out_shape=[
            jax.ShapeDtypeStruct((N, OUT), jnp.float32),
            jax.ShapeDtypeStruct((N, 2), jnp.float32),
        ],
    )(out1p.reshape(8, N, 128), ssum1t, b1.reshape(1, HEADS * HID), W2,
      a2cat)

    out2p, ssum2 = _sc_l2()(t2, esed2.T.reshape(2 * N), src, dst, zer,
                            zer1)
    ssum2t = ssum2.reshape(2, NP)[:, :N].T

    u, v = pl.pallas_call(
        _tc3_body,
        grid=(N // NBLK,),
        in_specs=[
            pl.BlockSpec((2, NBLK, OUT), lambda i: (0, i, 0)),
            pl.BlockSpec((NBLK, 2), lambda i: (i, 0)),
            pl.BlockSpec((1, OUT), lambda i: (0, 0)),
            pl.BlockSpec((2 * OUT, HID), lambda i: (0, 0)),
        ],
        out_specs=[
            pl.BlockSpec((NBLK, HID), lambda i: (i, 0)),
            pl.BlockSpec((NBLK, HID), lambda i: (i, 0)),
        ],
        out_shape=[
            jax.ShapeDtypeStruct((N, HID), jnp.float32),
            jax.ShapeDtypeStruct((N, HID), jnp.float32),
        ],
    )(out2p.reshape(2, N, 128), ssum2t, b2.reshape(1, OUT), Wp1)

    ef = _sc_dec()(u, v, row, col)

    inv = 1.0 / jnp.sqrt(jnp.float32(1.0 + 1e-5))
    s1 = (g1 * inv).reshape(1, HID)
    s2 = (g2 * inv).reshape(1, 32)
    out = pl.pallas_call(
        _tc4_body,
        grid=(EQ // MLP_BLK,),
        in_specs=[
            pl.BlockSpec((MLP_BLK, 2 * OUT), lambda i: (i, 0)),
            pl.BlockSpec((1, HID), lambda i: (0, 0)),
            pl.BlockSpec((1, HID), lambda i: (0, 0)),
            pl.BlockSpec((1, HID), lambda i: (0, 0)),
            pl.BlockSpec((HID, 32), lambda i: (0, 0)),
            pl.BlockSpec((1, 32), lambda i: (0, 0)),
            pl.BlockSpec((1, 32), lambda i: (0, 0)),
            pl.BlockSpec((1, 32), lambda i: (0, 0)),
            pl.BlockSpec((32, 1), lambda i: (0, 0)),
            pl.BlockSpec((1, 1), lambda i: (0, 0)),
        ],
        out_specs=pl.BlockSpec((MLP_BLK, 1), lambda i: (i, 0)),
        out_shape=jax.ShapeDtypeStruct((EQ, 1), jnp.float32),
    )(ef, bp1.reshape(1, HID), s1, be1.reshape(1, HID), Wp2,
      bp2.reshape(1, 32), s2, be2.reshape(1, 32), Wp3, bp3.reshape(1, 1))
    return out.reshape(EQ)


# speedup vs baseline: 26.5360x; 1.0410x over previous
import functools

import jax
import jax.numpy as jnp
from jax import lax
from jax.experimental import pallas as pl
from jax.experimental.pallas import tpu as pltpu
from jax.experimental.pallas import tpu_sc as plsc

N = 10000
NP = 10240
E = 320000
EQ = 65536
F_IN = 128
HID = 256
OUT = 128
HEADS = 4

NBLK = 400
MLP_BLK = 1024
K1 = 80
KD = 64
RPT = N // 16
CPT = NP // 16


def _tc1_body(x_ref, w1_ref, acat_ref, t1_ref, esed_ref):
    h1 = jnp.dot(x_ref[...], w1_ref[...], preferred_element_type=jnp.float32)
    for p in range(8):
        t1_ref[p] = h1[:, p * 128:(p + 1) * 128]
    esed_ref[...] = jnp.dot(h1, acat_ref[...],
                            preferred_element_type=jnp.float32)


def _tc2_body(outp_ref, ssum_ref, b1_ref, w2_ref, a2cat_ref, t2_ref,
              esed2_ref):
    acc = jnp.zeros((NBLK, OUT), jnp.float32)
    for p in range(8):
        h = p // 2
        denom = ssum_ref[:, h:h + 1] + 1e-16
        z = outp_ref[p] / denom + b1_ref[0:1, p * 128:(p + 1) * 128]
        z = jnp.where(z > 0, z, jnp.exp(jnp.minimum(z, 0.0)) - 1.0)
        acc = acc + jnp.dot(z, w2_ref[p * 128:(p + 1) * 128, :],
                            preferred_element_type=jnp.float32)
    t2_ref[...] = acc
    esed2_ref[...] = jnp.dot(acc, a2cat_ref[...],
                             preferred_element_type=jnp.float32)


def _tc3_body(out2p_ref, ssum2_ref, b2_ref, wp1_ref, u_ref, v_ref):
    denom = ssum2_ref[:, 0:1] + ssum2_ref[:, 1:2] + 1e-16
    z = (out2p_ref[0] + out2p_ref[1]) / denom + b2_ref[...]
    u_ref[...] = jnp.dot(z, wp1_ref[:OUT, :],
                         preferred_element_type=jnp.float32)
    v_ref[...] = jnp.dot(z, wp1_ref[OUT:, :],
                         preferred_element_type=jnp.float32)


def _tc4_body(ef_ref, bp1_ref, s1_ref, be1_ref, wp2_ref, bp2_ref, s2_ref,
              be2_ref, wp3_ref, bp3_ref, out_ref):
    h = (ef_ref[...] + bp1_ref[...]) * s1_ref[...] + be1_ref[...]
    h = jnp.maximum(h, 0.0)
    h2 = jnp.dot(h, wp2_ref[...], preferred_element_type=jnp.float32)
    h2 = (h2 + bp2_ref[...]) * s2_ref[...] + be2_ref[...]
    h2 = jnp.maximum(h2, 0.0)
    o = jnp.dot(h2, wp3_ref[...], preferred_element_type=jnp.float32)
    out_ref[...] = o + bp3_ref[...]


_SC_MESH = plsc.VectorSubcoreMesh(core_axis_name="c", subcore_axis_name="s")


def _zero_vec_ref(ref, nvec):
    z = jnp.zeros((16,), jnp.float32)

    def zb(i, _):
        ref[pl.ds(i * 16, 16)] = z
        return 0

    lax.fori_loop(0, nvec, zb, 0)


def _scale_rows(rows, pv, nrows, width_vecs):
    def sb(i, _):
        scv = plsc.load_gather(pv, [jnp.full((16,), 0, jnp.int32) + i])
        for v in range(width_vecs):
            sl = pl.ds(v * 16, 16)
            rows[i, sl] = rows[i, sl] * scv
        return 0

    lax.fori_loop(0, nrows, sb, 0)


def _edge_pass(nblk, do_ssum, base_e, tab_base, t_hbm, src_hbm, dst_hbm,
               es_v, ed_v, src_v, dst_v, pv, gidx, didx, rows,
               semb, semg, semsc, semp, acc_sh, ssum_sh):
    G = nblk * 25

    def stage_block(b, buf):
        e0 = base_e + b * 2000
        pltpu.async_copy(src_hbm.at[pl.ds(e0, 2000)], src_v.at[buf],
                         semb.at[buf])
        pltpu.async_copy(dst_hbm.at[pl.ds(e0, 2000)], dst_v.at[buf],
                         semb.at[buf])

    def wait_block(b, buf):
        e0 = base_e + b * 2000
        pltpu.make_async_copy(src_hbm.at[pl.ds(e0, 2000)], src_v.at[buf],
                              semb.at[buf]).wait()
        pltpu.make_async_copy(dst_hbm.at[pl.ds(e0, 2000)], dst_v.at[buf],
                              semb.at[buf]).wait()

    def scalars(gg, buf):
        bb = (gg // 25) % 2
        o = (gg % 25) * K1
        for j in range(K1 // 16):
            sl = pl.ds(o + j * 16, 16)
            bsl = pl.ds(j * 16, 16)
            s16 = src_v[bb, sl]
            d16 = dst_v[bb, sl]
            gidx[buf, bsl] = s16 + tab_base
            didx[buf, bsl] = d16
            ev = plsc.load_gather(es_v, [s16])
            dv = plsc.load_gather(ed_v, [d16])
            t = ev + dv
            e = jnp.where(t > 0, t, t * 0.2)
            pv[buf, bsl] = jnp.exp(e)

    def issue_gather(buf):
        pltpu.async_copy(t_hbm.at[gidx.at[buf]], rows.at[buf], semg.at[buf])

    def wait_gather(buf):
        pltpu.make_async_copy(t_hbm.at[gidx.at[buf]], rows.at[buf],
                              semg.at[buf]).wait()

    def issue_scatter(buf):
        pltpu.async_copy(rows.at[buf], acc_sh.at[didx.at[buf]],
                         semsc.at[buf], add=True)

        @pl.when(do_ssum)
        def _():
            pltpu.async_copy(pv.at[buf], ssum_sh.at[didx.at[buf]],
                             semp.at[buf], add=True)

    def wait_scatter(buf):
        pltpu.make_async_copy(rows.at[buf], acc_sh.at[didx.at[buf]],
                              semsc.at[buf]).wait()

        @pl.when(do_ssum)
        def _():
            pltpu.make_async_copy(pv.at[buf], ssum_sh.at[didx.at[buf]],
                                  semp.at[buf]).wait()

    def scale(buf):
        b16 = jnp.full((16,), 0, jnp.int32) + buf

        def sb(i, _):
            for u in range(4):
                iu = i * 4 + u
                i16 = jnp.full((16,), 0, jnp.int32) + iu
                scv = plsc.load_gather(pv, [b16, i16])
                for v in range(8):
                    sl = pl.ds(v * 16, 16)
                    rows[buf, iu, sl] = rows[buf, iu, sl] * scv
            return 0

        lax.fori_loop(0, K1 // 4, sb, 0)

    stage_block(0, 0)
    wait_block(0, 0)
    if nblk > 1:
        stage_block(1, 1)
    scalars(0, 0)
    issue_gather(0)

    def it(g, _):
        buf = g % 2
        buf2 = (g + 1) % 2
        wait_gather(buf)

        @pl.when(g + 1 < G)
        def _():
            @pl.when((g + 1) % 25 == 0)
            def _():
                b = (g + 1) // 25
                wait_block(b, b % 2)

                @pl.when(b + 1 < nblk)
                def _():
                    stage_block(b + 1, (b + 1) % 2)

            @pl.when(g >= 1)
            def _():
                wait_scatter(buf2)

            scalars(g + 1, buf2)
            issue_gather(buf2)

        scale(buf)
        issue_scatter(buf)
        return 0

    lax.fori_loop(0, G, it, 0)
    wait_scatter(0)
    wait_scatter(1)


def _sc_l1_body(t1_hbm, esed_hbm, src_hbm, dst_hbm, zer_hbm, zer1_hbm,
                out_hbm, ssum_hbm,
                es_v, ed_v, src_v, dst_v, pv, gidx, didx, rows,
                semb, semg, semsc, semp, acc_sh, ssum_sh):
    c = lax.axis_index("c")
    s = lax.axis_index("s")

    def one_pass(pp, _):
        p = c * 4 + pp
        h = c * 2 + pp // 2
        even = (pp % 2) == 0
        pltpu.sync_copy(esed_hbm.at[pl.ds(h * N, N)], es_v)
        pltpu.sync_copy(esed_hbm.at[pl.ds((4 + h) * N, N)], ed_v)
        pltpu.sync_copy(zer_hbm, acc_sh.at[pl.ds(s * RPT, RPT)])

        @pl.when(even)
        def _():
            pltpu.sync_copy(zer1_hbm, ssum_sh.at[pl.ds(s * CPT, CPT)])

        plsc.subcore_barrier()
        _edge_pass(10, even, s * 20000, p * N, t1_hbm, src_hbm, dst_hbm,
                   es_v, ed_v, src_v, dst_v, pv, gidx, didx, rows,
                   semb, semg, semsc, semp, acc_sh, ssum_sh)
        plsc.subcore_barrier()
        pltpu.sync_copy(acc_sh.at[pl.ds(s * RPT, RPT)],
                        out_hbm.at[pl.ds(p * N + s * RPT, RPT)])

        @pl.when(even)
        def _():
            pltpu.sync_copy(ssum_sh.at[pl.ds(s * CPT, CPT)],
                            ssum_hbm.at[pl.ds(h * NP + s * CPT, CPT)])

        plsc.subcore_barrier()
        return 0

    lax.fori_loop(0, 4, one_pass, 0)


def _sc_l2_body(t2_hbm, esed2_hbm, src_hbm, dst_hbm, zer_hbm, zer1_hbm,
                out_hbm, ssum_hbm,
                es_v, ed_v, src_v, dst_v, pv, gidx, didx, rows,
                semb, semg, semsc, semp, acc_sh, ssum_sh):
    c = lax.axis_index("c")
    s = lax.axis_index("s")
    pltpu.sync_copy(esed2_hbm.at[pl.ds(0, N)], es_v)
    pltpu.sync_copy(esed2_hbm.at[pl.ds(N, N)], ed_v)
    pltpu.sync_copy(zer_hbm, acc_sh.at[pl.ds(s * RPT, RPT)])
    pltpu.sync_copy(zer1_hbm, ssum_sh.at[pl.ds(s * CPT, CPT)])
    plsc.subcore_barrier()
    _edge_pass(5, s >= 0, (c * 16 + s) * 10000, 0, t2_hbm, src_hbm, dst_hbm,
               es_v, ed_v, src_v, dst_v, pv, gidx, didx, rows,
               semb, semg, semsc, semp, acc_sh, ssum_sh)
    plsc.subcore_barrier()
    pltpu.sync_copy(acc_sh.at[pl.ds(s * RPT, RPT)],
                    out_hbm.at[pl.ds(c * N + s * RPT, RPT)])
    pltpu.sync_copy(ssum_sh.at[pl.ds(s * CPT, CPT)],
                    ssum_hbm.at[pl.ds(c * NP + s * CPT, CPT)])


def _sc_dec_body(u_hbm, v_hbm, row_hbm, col_hbm, ef_hbm,
                 row_v, col_v, ru, rv, semu, semv):
    c = lax.axis_index("c")
    s = lax.axis_index("s")
    tid = c * 16 + s
    q0 = tid * (EQ // 32)
    pltpu.sync_copy(row_hbm.at[pl.ds(q0, EQ // 32)], row_v)
    pltpu.sync_copy(col_hbm.at[pl.ds(q0, EQ // 32)], col_v)

    def one_batch(kb, _):
        o = kb * KD
        cu = pltpu.async_copy(u_hbm.at[row_v.at[pl.ds(o, KD)]], ru, semu)
        cv = pltpu.async_copy(v_hbm.at[col_v.at[pl.ds(o, KD)]], rv, semv)
        cu.wait()
        cv.wait()

        def add_row(i, _):
            for v in range(16):
                sl = pl.ds(v * 16, 16)
                ru[i, sl] = ru[i, sl] + rv[i, sl]
            return 0

        lax.fori_loop(0, KD, add_row, 0)
        pltpu.sync_copy(ru, ef_hbm.at[pl.ds(q0 + o, KD)])
        return 0

    lax.fori_loop(0, (EQ // 32) // KD, one_batch, 0)


_sc_l1 = functools.partial(
    pl.kernel, _sc_l1_body,
    out_type=(jax.ShapeDtypeStruct((8 * N, 128), jnp.float32),
              jax.ShapeDtypeStruct((4 * NP,), jnp.float32)),
    mesh=_SC_MESH,
    compiler_params=pltpu.CompilerParams(use_tc_tiling_on_sc=False, needs_layout_passes=False),
    scratch_types=[
        pltpu.VMEM((N,), jnp.float32),
        pltpu.VMEM((N,), jnp.float32),
        pltpu.VMEM((2, 2000), jnp.int32),
        pltpu.VMEM((2, 2000), jnp.int32),
        pltpu.VMEM((2, K1), jnp.float32),
        pltpu.VMEM((2, K1), jnp.int32),
        pltpu.VMEM((2, K1), jnp.int32),
        pltpu.VMEM((2, K1, 128), jnp.float32),
        pltpu.SemaphoreType.DMA((2,)),
        pltpu.SemaphoreType.DMA((2,)),
        pltpu.SemaphoreType.DMA((2,)),
        pltpu.SemaphoreType.DMA((2,)),
        pltpu.VMEM_SHARED((N, 128), jnp.float32),
        pltpu.VMEM_SHARED((NP,), jnp.float32),
    ])

_sc_l2 = functools.partial(
    pl.kernel, _sc_l2_body,
    out_type=(jax.ShapeDtypeStruct((2 * N, 128), jnp.float32),
              jax.ShapeDtypeStruct((2 * NP,), jnp.float32)),
    mesh=_SC_MESH,
    compiler_params=pltpu.CompilerParams(use_tc_tiling_on_sc=False, needs_layout_passes=False),
    scratch_types=[
        pltpu.VMEM((N,), jnp.float32),
        pltpu.VMEM((N,), jnp.float32),
        pltpu.VMEM((2, 2000), jnp.int32),
        pltpu.VMEM((2, 2000), jnp.int32),
        pltpu.VMEM((2, K1), jnp.float32),
        pltpu.VMEM((2, K1), jnp.int32),
        pltpu.VMEM((2, K1), jnp.int32),
        pltpu.VMEM((2, K1, 128), jnp.float32),
        pltpu.SemaphoreType.DMA((2,)),
        pltpu.SemaphoreType.DMA((2,)),
        pltpu.SemaphoreType.DMA((2,)),
        pltpu.SemaphoreType.DMA((2,)),
        pltpu.VMEM_SHARED((N, 128), jnp.float32),
        pltpu.VMEM_SHARED((NP,), jnp.float32),
    ])

_sc_dec = functools.partial(
    pl.kernel, _sc_dec_body,
    out_type=jax.ShapeDtypeStruct((EQ, 2 * OUT), jnp.float32),
    mesh=_SC_MESH,
    compiler_params=pltpu.CompilerParams(use_tc_tiling_on_sc=False, needs_layout_passes=False),
    scratch_types=[
        pltpu.VMEM((EQ // 32,), jnp.int32),
        pltpu.VMEM((EQ // 32,), jnp.int32),
        pltpu.VMEM((KD, 2 * OUT), jnp.float32),
        pltpu.VMEM((KD, 2 * OUT), jnp.float32),
        pltpu.SemaphoreType.DMA,
        pltpu.SemaphoreType.DMA,
    ])


def kernel(x, edge_index, edge_label_index, W1, a1s, a1d, b1, W2, a2s, a2d,
           b2, Wp1, bp1, g1, be1, Wp2, bp2, g2, be2, Wp3, bp3):
    src = edge_index[0].astype(jnp.int32)
    dst = edge_index[1].astype(jnp.int32)
    row = edge_label_index[0].astype(jnp.int32)
    col = edge_label_index[1].astype(jnp.int32)

    eye4 = jnp.eye(HEADS, dtype=jnp.float32)
    asb = (eye4[:, None, :] * a1s[:, :, None]).reshape(HEADS * HID, HEADS)
    adb = (eye4[:, None, :] * a1d[:, :, None]).reshape(HEADS * HID, HEADS)
    acat = jnp.concatenate([asb, adb], axis=1)
    a2cat = jnp.concatenate([a2s, a2d], axis=0).T
    zer = jnp.zeros((RPT, 128), jnp.float32)
    zer1 = jnp.zeros((CPT,), jnp.float32)

    t1, esed1 = pl.pallas_call(
        _tc1_body,
        grid=(N // NBLK,),
        in_specs=[
            pl.BlockSpec((NBLK, F_IN), lambda i: (i, 0)),
            pl.BlockSpec((F_IN, HEADS * HID), lambda i: (0, 0)),
            pl.BlockSpec((HEADS * HID, 8), lambda i: (0, 0)),
        ],
        out_specs=[
            pl.BlockSpec((8, NBLK, 128), lambda i: (0, i, 0)),
            pl.BlockSpec((NBLK, 8), lambda i: (i, 0)),
        ],
        out_shape=[
            jax.ShapeDtypeStruct((8, N, 128), jnp.float32),
            jax.ShapeDtypeStruct((N, 8), jnp.float32),
        ],
    )(x, W1, acat)

    out1p, ssum1 = _sc_l1()(t1.reshape(8 * N, 128), esed1.T.reshape(8 * N),
                            src, dst, zer, zer1)
    ssum1t = ssum1.reshape(4, NP)[:, :N].T

    t2, esed2 = pl.pallas_call(
        _tc2_body,
        grid=(N // NBLK,),
        in_specs=[
            pl.BlockSpec((8, NBLK, 128), lambda i: (0, i, 0)),
            pl.BlockSpec((NBLK, HEADS), lambda i: (i, 0)),
            pl.BlockSpec((1, HEADS * HID), lambda i: (0, 0)),
            pl.BlockSpec((HEADS * HID, OUT), lambda i: (0, 0)),
            pl.BlockSpec((OUT, 2), lambda i: (0, 0)),
        ],
        out_specs=[
            pl.BlockSpec((NBLK, OUT), lambda i: (i, 0)),
            pl.BlockSpec((NBLK, 2), lambda i: (i, 0)),
        ],
        out_shape=[
            jax.ShapeDtypeStruct((N, OUT), jnp.float32),
            jax.ShapeDtypeStruct((N, 2), jnp.float32),
        ],
    )(out1p.reshape(8, N, 128), ssum1t, b1.reshape(1, HEADS * HID), W2,
      a2cat)

    out2p, ssum2 = _sc_l2()(t2, esed2.T.reshape(2 * N), src, dst, zer,
                            zer1)
    ssum2t = ssum2.reshape(2, NP)[:, :N].T

    u, v = pl.pallas_call(
        _tc3_body,
        grid=(N // NBLK,),
        in_specs=[
            pl.BlockSpec((2, NBLK, OUT), lambda i: (0, i, 0)),
            pl.BlockSpec((NBLK, 2), lambda i: (i, 0)),
            pl.BlockSpec((1, OUT), lambda i: (0, 0)),
            pl.BlockSpec((2 * OUT, HID), lambda i: (0, 0)),
        ],
        out_specs=[
            pl.BlockSpec((NBLK, HID), lambda i: (i, 0)),
            pl.BlockSpec((NBLK, HID), lambda i: (i, 0)),
        ],
        out_shape=[
            jax.ShapeDtypeStruct((N, HID), jnp.float32),
            jax.ShapeDtypeStruct((N, HID), jnp.float32),
        ],
    )(out2p.reshape(2, N, 128), ssum2t, b2.reshape(1, OUT), Wp1)

    ef = _sc_dec()(u, v, row, col)

    inv = 1.0 / jnp.sqrt(jnp.float32(1.0 + 1e-5))
    s1 = (g1 * inv).reshape(1, HID)
    s2 = (g2 * inv).reshape(1, 32)
    out = pl.pallas_call(
        _tc4_body,
        grid=(EQ // MLP_BLK,),
        in_specs=[
            pl.BlockSpec((MLP_BLK, 2 * OUT), lambda i: (i, 0)),
            pl.BlockSpec((1, HID), lambda i: (0, 0)),
            pl.BlockSpec((1, HID), lambda i: (0, 0)),
            pl.BlockSpec((1, HID), lambda i: (0, 0)),
            pl.BlockSpec((HID, 32), lambda i: (0, 0)),
            pl.BlockSpec((1, 32), lambda i: (0, 0)),
            pl.BlockSpec((1, 32), lambda i: (0, 0)),
            pl.BlockSpec((1, 32), lambda i: (0, 0)),
            pl.BlockSpec((32, 1), lambda i: (0, 0)),
            pl.BlockSpec((1, 1), lambda i: (0, 0)),
        ],
        out_specs=pl.BlockSpec((MLP_BLK, 1), lambda i: (i, 0)),
        out_shape=jax.ShapeDtypeStruct((EQ, 1), jnp.float32),
    )(ef, bp1.reshape(1, HID), s1, be1.reshape(1, HID), Wp2,
      bp2.reshape(1, 32), s2, be2.reshape(1, 32), Wp3, bp3.reshape(1, 1))
    return out.reshape(EQ)
